# baseline (speedup 1.0000x reference)
import functools

import jax
import jax.numpy as jnp
from jax import lax
from jax.experimental import pallas as pl
from jax.experimental.pallas import tpu as pltpu
from jax.experimental.pallas import tpu_sc as plsc


def _sc_gather(gammas, ts):
    mesh = plsc.VectorSubcoreMesh(core_axis_name="c", subcore_axis_name="s")

    @functools.partial(
        pl.kernel,
        mesh=mesh,
        out_type=jax.ShapeDtypeStruct((32,), jnp.float32),
        scratch_types=[
            pltpu.VMEM((16,), jnp.int32),
            pltpu.VMEM((16,), jnp.float32),
            pltpu.SemaphoreType.DMA,
        ],
    )
    def k(g_hbm, t_hbm, out_hbm, idx_v, rows_v, sem):
        wid = lax.axis_index("s") * 2 + lax.axis_index("c")

        @pl.when(wid < 2)
        def _():
            base = wid * 16
            pltpu.sync_copy(t_hbm.at[pl.ds(base, 16)], idx_v)
            pltpu.async_copy(g_hbm.at[idx_v], rows_v, sem).wait()
            pltpu.sync_copy(rows_v, out_hbm.at[pl.ds(base, 16)])

    return k(gammas, ts)


def _tc_body(g_ref, x_ref, n_ref, o_ref):
    b = pl.program_id(0)
    g = g_ref[b]
    o_ref[...] = jnp.sqrt(g) * x_ref[...] + jnp.sqrt(1.0 - g) * n_ref[...]


def kernel(x_start, timesteps, noise, gammas):
    B, C, H, W = x_start.shape
    ts = timesteps.reshape(B).astype(jnp.int32)
    gvals = _sc_gather(gammas.astype(jnp.float32), ts)

    grid_spec = pltpu.PrefetchScalarGridSpec(
        num_scalar_prefetch=1,
        grid=(B,),
        in_specs=[
            pl.BlockSpec((1, C, H, W), lambda b, g: (b, 0, 0, 0)),
            pl.BlockSpec((1, C, H, W), lambda b, g: (b, 0, 0, 0)),
        ],
        out_specs=pl.BlockSpec((1, C, H, W), lambda b, g: (b, 0, 0, 0)),
    )
    return pl.pallas_call(
        _tc_body,
        grid_spec=grid_spec,
        out_shape=jax.ShapeDtypeStruct((B, C, H, W), jnp.float32),
    )(gvals, x_start, noise)
